# Initial kernel scaffold; baseline (speedup 1.0000x reference)
import jax
import jax.numpy as jnp
from jax.experimental import pallas as pl


def kernel(x, edge_index, reranker_scores, W_l, b_l, W_r, W_s, b_s, alpha):
    raise NotImplementedError("write your pallas kernel here")


# speedup vs baseline: 3.6185x; 3.6185x over previous
import functools

import jax
import jax.numpy as jnp
from jax import lax
from jax.experimental import pallas as pl
from jax.experimental.pallas import tpu as pltpu, tpu_sc as plsc

N = 10000
E = 320000
D = 128
H = 128
DH = D // 2

NC = 2
NS = 16
NW = NC * NS
CHUNK = 128
NCHUNKS = 80
KB = 4
EPW = CHUNK * NCHUNKS
EPAD = NW * EPW
NPAD = 10240
ROWS_PER_TILE = NPAD // NS
DUMP_ROW = N + 16
VPC = CHUNK // 16


def _sc_segment_sum(x_lo, x_hi, src_w, dst_w):
    mesh = plsc.VectorSubcoreMesh(
        core_axis_name="c", subcore_axis_name="s", num_cores=NC, num_subcores=NS
    )

    @functools.partial(
        pl.kernel,
        out_type=(
            jax.ShapeDtypeStruct((NC, NPAD, DH), jnp.float32),
            jax.ShapeDtypeStruct((NC, NPAD, DH), jnp.float32),
            jax.ShapeDtypeStruct((NW, NPAD), jnp.float32),
        ),
        mesh=mesh,
        compiler_params=pltpu.CompilerParams(needs_layout_passes=False,
                                             use_tc_tiling_on_sc=False),
        scratch_types=[
            pltpu.VMEM((NCHUNKS, CHUNK), jnp.int32),
            pltpu.VMEM((NCHUNKS, CHUNK), jnp.int32),
            [pltpu.VMEM((CHUNK, DH), jnp.float32)] * KB,
            pltpu.VMEM((CHUNK, DH), jnp.float32),
            pltpu.VMEM((NPAD,), jnp.float32),
            pltpu.VMEM_SHARED((NPAD, DH), jnp.float32),
            [pltpu.SemaphoreType.DMA] * KB,
        ],
    )
    def seg_kernel(xlo_hbm, xhi_hbm, src_hbm, dst_hbm, lo_out, hi_out, cnt_out,
                   src_v, dst_v, bufs, zbuf, cnt_v, acc_sh, sems):
        c = lax.axis_index("c")
        s = lax.axis_index("s")
        wid = c * NS + s
        base = s * ROWS_PER_TILE

        pltpu.sync_copy(src_hbm.at[wid], src_v)
        pltpu.sync_copy(dst_hbm.at[wid], dst_v)

        z16 = jnp.zeros((16,), jnp.float32)

        def zero_zbuf(i, _):
            zbuf[i // (DH // 16), pl.ds((i % (DH // 16)) * 16, 16)] = z16
            return 0
        lax.fori_loop(0, CHUNK * (DH // 16), zero_zbuf, 0)

        def zero_cnt(i, _):
            cnt_v[pl.ds(i * 16, 16)] = z16
            return 0
        lax.fori_loop(0, NPAD // 16, zero_cnt, 0)

        def zero_stripe():
            for k in range(ROWS_PER_TILE // CHUNK):
                pltpu.sync_copy(zbuf, acc_sh.at[pl.ds(base + k * CHUNK, CHUNK)])

        def copy_stripe_out(out_ref):
            pltpu.sync_copy(acc_sh.at[pl.ds(base, ROWS_PER_TILE)],
                            out_ref.at[c, pl.ds(base, ROWS_PER_TILE)])

        ones16 = jnp.ones((16,), jnp.float32)

        def edge_pass(x_hbm, with_hist):
            def outer(gg, _):
                j0 = gg * KB
                for b in range(KB):
                    pltpu.async_copy(x_hbm.at[src_v.at[j0 + b]], bufs[b], sems[b])
                if with_hist:
                    for b in range(KB):
                        for v in range(VPC):
                            idx = dst_v[j0 + b, pl.ds(v * 16, 16)]
                            plsc.addupdate_scatter(cnt_v, [idx], ones16)
                for b in range(KB):
                    pltpu.make_async_copy(
                        x_hbm.at[src_v.at[j0 + b]], bufs[b], sems[b]).wait()
                    pltpu.sync_copy(bufs[b], acc_sh.at[dst_v.at[j0 + b]],
                                    add=True)
                return 0
            lax.fori_loop(0, NCHUNKS // KB, outer, 0)

        zero_stripe()
        plsc.subcore_barrier()
        edge_pass(xlo_hbm, True)
        pltpu.sync_copy(cnt_v, cnt_out.at[wid])
        plsc.subcore_barrier()
        copy_stripe_out(lo_out)

        zero_stripe()
        plsc.subcore_barrier()
        edge_pass(xhi_hbm, False)
        plsc.subcore_barrier()
        copy_stripe_out(hi_out)

    return seg_kernel(x_lo, x_hi, src_w, dst_w)


def _tc_combine(x, lo, hi, cnts, rr, Wl_lo, Wl_hi, W_r, W_sT, b_l2, misc):
    R = 400
    G = N // R

    def body(x_ref, lo_ref, hi_ref, cnt_ref, rr_ref, wllo_ref, wlhi_ref,
             wr_ref, ws_ref, bl_ref, misc_ref, out_ref):
        x = x_ref[...]
        cnt = jnp.sum(cnt_ref[0], axis=0).reshape(R, 1)
        rcnt = 1.0 / jnp.maximum(cnt, 1.0)
        mean_lo = (lo_ref[0] + lo_ref[1]) * rcnt
        mean_hi = (hi_ref[0] + hi_ref[1]) * rcnt
        z = (lax.dot_general(mean_lo, wllo_ref[...], (((1,), (1,)), ((), ())),
                             preferred_element_type=jnp.float32)
             + lax.dot_general(mean_hi, wlhi_ref[...], (((1,), (1,)), ((), ())),
                               preferred_element_type=jnp.float32)
             + bl_ref[...]
             + lax.dot_general(x, wr_ref[...], (((1,), (1,)), ((), ())),
                               preferred_element_type=jnp.float32))
        h = jnp.maximum(z, 0.0) + x
        gnn = lax.dot_general(h, ws_ref[...], (((1,), (0,)), ((), ())),
                              preferred_element_type=jnp.float32)
        b_s = misc_ref[0, 0]
        a = 1.0 / (1.0 + jnp.exp(-misc_ref[0, 1]))
        out_ref[...] = a * rr_ref[...] + (1.0 - a) * (gnn + b_s)

    return pl.pallas_call(
        body,
        grid=(G,),
        in_specs=[
            pl.BlockSpec((R, D), lambda i: (i, 0)),
            pl.BlockSpec((NC, R, DH), lambda i: (0, i, 0)),
            pl.BlockSpec((NC, R, DH), lambda i: (0, i, 0)),
            pl.BlockSpec((1, NW, R), lambda i: (i, 0, 0)),
            pl.BlockSpec((R, 1), lambda i: (i, 0)),
            pl.BlockSpec((H, DH), lambda i: (0, 0)),
            pl.BlockSpec((H, DH), lambda i: (0, 0)),
            pl.BlockSpec((H, D), lambda i: (0, 0)),
            pl.BlockSpec((H, 1), lambda i: (0, 0)),
            pl.BlockSpec((1, H), lambda i: (0, 0)),
            pl.BlockSpec((1, 2), lambda i: (0, 0)),
        ],
        out_specs=pl.BlockSpec((R, 1), lambda i: (i, 0)),
        out_shape=jax.ShapeDtypeStruct((N, 1), jnp.float32),
    )(x, lo, hi, cnts, rr, Wl_lo, Wl_hi, W_r, W_sT, b_l2, misc)


def kernel(x, edge_index, reranker_scores, W_l, b_l, W_r, W_s, b_s, alpha):
    src = jnp.zeros((EPAD,), jnp.int32).at[:E].set(edge_index[0])
    dst = jnp.full((EPAD,), DUMP_ROW, jnp.int32).at[:E].set(edge_index[1])
    src_w = src.reshape(NW, NCHUNKS, CHUNK)
    dst_w = dst.reshape(NW, NCHUNKS, CHUNK)
    lo, hi, cnts = _sc_segment_sum(x[:, :DH], x[:, DH:], src_w, dst_w)
    cnts = cnts[:, :N].reshape(NW, N // 400, 400).transpose(1, 0, 2)
    misc = jnp.stack([b_s[0], alpha]).reshape(1, 2)
    out = _tc_combine(x, lo, hi, cnts, reranker_scores.reshape(N, 1),
                      W_l[:, :DH], W_l[:, DH:], W_r, W_s.reshape(H, 1),
                      b_l.reshape(1, H), misc)
    return out[:, 0]


# speedup vs baseline: 6.4380x; 1.7792x over previous
import functools

import jax
import jax.numpy as jnp
from jax import lax
from jax.experimental import pallas as pl
from jax.experimental.pallas import tpu as pltpu, tpu_sc as plsc

N = 10000
E = 320000
D = 128
H = 128

NC = 2
NS = 16
NW = NC * NS
CHUNK = 128
NCHUNKS = 80
KB = 4
EPW = CHUNK * NCHUNKS
EPAD = NW * EPW
NPAD = 10240
ROWS_PER_TILE = NPAD // NS
DUMP_ROW = N + 16
VPC = CHUNK // 16


def _sc_segment_sum(x_bf, src_w, dst_w):
    mesh = plsc.VectorSubcoreMesh(
        core_axis_name="c", subcore_axis_name="s", num_cores=NC, num_subcores=NS
    )

    @functools.partial(
        pl.kernel,
        out_type=(
            jax.ShapeDtypeStruct((NC, NPAD, D), jnp.bfloat16),
            jax.ShapeDtypeStruct((NW, NPAD), jnp.float32),
        ),
        mesh=mesh,
        compiler_params=pltpu.CompilerParams(needs_layout_passes=False,
                                             use_tc_tiling_on_sc=False),
        scratch_types=[
            pltpu.VMEM((NCHUNKS, CHUNK), jnp.int32),
            pltpu.VMEM((NCHUNKS, CHUNK), jnp.int32),
            [pltpu.VMEM((CHUNK, D), jnp.bfloat16)] * KB,
            pltpu.VMEM((CHUNK, D), jnp.bfloat16),
            pltpu.VMEM((NPAD,), jnp.float32),
            pltpu.VMEM_SHARED((NPAD, D), jnp.bfloat16),
            [pltpu.SemaphoreType.DMA] * KB,
        ],
    )
    def seg_kernel(x_hbm, src_hbm, dst_hbm, sum_out, cnt_out,
                   src_v, dst_v, bufs, zbuf, cnt_v, acc_sh, sems):
        c = lax.axis_index("c")
        s = lax.axis_index("s")
        wid = c * NS + s
        base = s * ROWS_PER_TILE

        pltpu.sync_copy(src_hbm.at[wid], src_v)
        pltpu.sync_copy(dst_hbm.at[wid], dst_v)

        z32b = jnp.zeros((32,), jnp.bfloat16)

        def zero_zbuf(i, _):
            zbuf[i // (D // 32), pl.ds((i % (D // 32)) * 32, 32)] = z32b
            return 0
        lax.fori_loop(0, CHUNK * (D // 32), zero_zbuf, 0)

        z16 = jnp.zeros((16,), jnp.float32)

        def zero_cnt(i, _):
            cnt_v[pl.ds(i * 16, 16)] = z16
            return 0
        lax.fori_loop(0, NPAD // 16, zero_cnt, 0)

        for k in range(ROWS_PER_TILE // CHUNK):
            pltpu.sync_copy(zbuf, acc_sh.at[pl.ds(base + k * CHUNK, CHUNK)])

        plsc.subcore_barrier()

        ones16 = jnp.ones((16,), jnp.float32)

        def outer(gg, _):
            j0 = gg * KB
            for b in range(KB):
                pltpu.async_copy(x_hbm.at[src_v.at[j0 + b]], bufs[b], sems[b])
            for b in range(KB):
                for v in range(VPC):
                    idx = dst_v[j0 + b, pl.ds(v * 16, 16)]
                    plsc.addupdate_scatter(cnt_v, [idx], ones16)
            for b in range(KB):
                pltpu.make_async_copy(
                    x_hbm.at[src_v.at[j0 + b]], bufs[b], sems[b]).wait()
                pltpu.sync_copy(bufs[b], acc_sh.at[dst_v.at[j0 + b]], add=True)
            return 0
        lax.fori_loop(0, NCHUNKS // KB, outer, 0)

        pltpu.sync_copy(cnt_v, cnt_out.at[wid])

        plsc.subcore_barrier()

        pltpu.sync_copy(acc_sh.at[pl.ds(base, ROWS_PER_TILE)],
                        sum_out.at[c, pl.ds(base, ROWS_PER_TILE)])

    return seg_kernel(x_bf, src_w, dst_w)


def _tc_combine(x, sums, cnts, rr, W_l, W_r, W_sT, b_l2, misc):
    R = 400
    G = N // R

    def body(x_ref, sum_ref, cnt_ref, rr_ref, wl_ref, wr_ref, ws_ref,
             bl_ref, misc_ref, out_ref):
        x = x_ref[...]
        agg = (sum_ref[0].astype(jnp.float32)
               + sum_ref[1].astype(jnp.float32))
        cnt = jnp.sum(cnt_ref[0], axis=0).reshape(R, 1)
        mean = agg / jnp.maximum(cnt, 1.0)
        z = (lax.dot_general(mean, wl_ref[...], (((1,), (1,)), ((), ())),
                             preferred_element_type=jnp.float32)
             + bl_ref[...]
             + lax.dot_general(x, wr_ref[...], (((1,), (1,)), ((), ())),
                               preferred_element_type=jnp.float32))
        h = jnp.maximum(z, 0.0) + x
        gnn = lax.dot_general(h, ws_ref[...], (((1,), (0,)), ((), ())),
                              preferred_element_type=jnp.float32)
        b_s = misc_ref[0, 0]
        a = 1.0 / (1.0 + jnp.exp(-misc_ref[0, 1]))
        out_ref[...] = a * rr_ref[...] + (1.0 - a) * (gnn + b_s)

    return pl.pallas_call(
        body,
        grid=(G,),
        in_specs=[
            pl.BlockSpec((R, D), lambda i: (i, 0)),
            pl.BlockSpec((NC, R, D), lambda i: (0, i, 0)),
            pl.BlockSpec((1, NW, R), lambda i: (i, 0, 0)),
            pl.BlockSpec((R, 1), lambda i: (i, 0)),
            pl.BlockSpec((H, D), lambda i: (0, 0)),
            pl.BlockSpec((H, D), lambda i: (0, 0)),
            pl.BlockSpec((H, 1), lambda i: (0, 0)),
            pl.BlockSpec((1, H), lambda i: (0, 0)),
            pl.BlockSpec((1, 2), lambda i: (0, 0)),
        ],
        out_specs=pl.BlockSpec((R, 1), lambda i: (i, 0)),
        out_shape=jax.ShapeDtypeStruct((N, 1), jnp.float32),
    )(x, sums, cnts, rr, W_l, W_r, W_sT, b_l2, misc)


def kernel(x, edge_index, reranker_scores, W_l, b_l, W_r, W_s, b_s, alpha):
    x_bf = x.astype(jnp.bfloat16)
    src = jnp.zeros((EPAD,), jnp.int32).at[:E].set(edge_index[0])
    dst = jnp.full((EPAD,), DUMP_ROW, jnp.int32).at[:E].set(edge_index[1])
    src_w = src.reshape(NW, NCHUNKS, CHUNK)
    dst_w = dst.reshape(NW, NCHUNKS, CHUNK)
    sums, cnts = _sc_segment_sum(x_bf, src_w, dst_w)
    cnts = cnts[:, :N].reshape(NW, N // 400, 400).transpose(1, 0, 2)
    misc = jnp.stack([b_s[0], alpha]).reshape(1, 2)
    out = _tc_combine(x, sums, cnts, reranker_scores.reshape(N, 1), W_l, W_r,
                      W_s.reshape(H, 1), b_l.reshape(1, H), misc)
    return out[:, 0]


# speedup vs baseline: 11.5189x; 1.7892x over previous
import functools

import jax
import jax.numpy as jnp
from jax import lax
from jax.experimental import pallas as pl
from jax.experimental.pallas import tpu as pltpu, tpu_sc as plsc

N = 10000
E = 320000
D = 128
H = 128

NC = 2
NS = 16
NW = NC * NS
CHUNK = 128
NCHUNKS = 80
KB = 4
EPW = CHUNK * NCHUNKS
EPAD = NW * EPW
NPAD = 10240
ROWS_PER_TILE = NPAD // NS
DUMP_ROW = N + 16
VPC = CHUNK // 16


def _sc_segment_sum(x_bf, src_w, dst_w):
    mesh = plsc.VectorSubcoreMesh(
        core_axis_name="c", subcore_axis_name="s", num_cores=NC, num_subcores=NS
    )

    @functools.partial(
        pl.kernel,
        out_type=(
            jax.ShapeDtypeStruct((NC, NPAD, D), jnp.bfloat16),
            jax.ShapeDtypeStruct((NW, NPAD), jnp.float32),
        ),
        mesh=mesh,
        compiler_params=pltpu.CompilerParams(needs_layout_passes=False,
                                             use_tc_tiling_on_sc=False),
        scratch_types=[
            pltpu.VMEM((NCHUNKS, CHUNK), jnp.int32),
            pltpu.VMEM((NCHUNKS, CHUNK), jnp.int32),
            [pltpu.VMEM((CHUNK, D), jnp.bfloat16)] * KB,
            pltpu.VMEM((CHUNK, D), jnp.bfloat16),
            pltpu.VMEM((NPAD,), jnp.float32),
            pltpu.VMEM_SHARED((NPAD, D), jnp.bfloat16),
            [pltpu.SemaphoreType.DMA] * KB,
        ],
    )
    def seg_kernel(x_hbm, src_hbm, dst_hbm, sum_out, cnt_out,
                   src_v, dst_v, bufs, zbuf, cnt_v, acc_sh, sems):
        c = lax.axis_index("c")
        s = lax.axis_index("s")
        wid = c * NS + s
        base = s * ROWS_PER_TILE

        pltpu.sync_copy(src_hbm.at[wid], src_v)
        pltpu.sync_copy(dst_hbm.at[wid], dst_v)

        z32b = jnp.zeros((32,), jnp.bfloat16)

        def zero_zbuf(i, _):
            zbuf[i // (D // 32), pl.ds((i % (D // 32)) * 32, 32)] = z32b
            return 0
        lax.fori_loop(0, CHUNK * (D // 32), zero_zbuf, 0)

        z16 = jnp.zeros((16,), jnp.float32)

        def zero_cnt(i, _):
            cnt_v[pl.ds(i * 16, 16)] = z16
            return 0
        lax.fori_loop(0, NPAD // 16, zero_cnt, 0)

        for k in range(ROWS_PER_TILE // CHUNK):
            pltpu.sync_copy(zbuf, acc_sh.at[pl.ds(base + k * CHUNK, CHUNK)])

        plsc.subcore_barrier()

        ones16 = jnp.ones((16,), jnp.float32)

        def outer(gg, _):
            j0 = gg * KB
            for b in range(KB):
                pltpu.async_copy(x_hbm.at[src_v.at[j0 + b]], bufs[b], sems[b])
            for b in range(KB):
                for v in range(VPC):
                    idx = dst_v[j0 + b, pl.ds(v * 16, 16)]
                    plsc.addupdate_scatter(cnt_v, [idx], ones16)
            for b in range(KB):
                pltpu.make_async_copy(
                    x_hbm.at[src_v.at[j0 + b]], bufs[b], sems[b]).wait()
                pltpu.sync_copy(bufs[b], acc_sh.at[dst_v.at[j0 + b]], add=True)
            return 0
        lax.fori_loop(0, NCHUNKS // KB, outer, 0)

        pltpu.sync_copy(cnt_v, cnt_out.at[wid])

        plsc.subcore_barrier()

        pltpu.sync_copy(acc_sh.at[pl.ds(base, ROWS_PER_TILE)],
                        sum_out.at[c, pl.ds(base, ROWS_PER_TILE)])

    return seg_kernel(x_bf, src_w, dst_w)


def _tc_combine(x, sums, cnts, rr, W_l, W_r, W_sT, b_l2, misc):
    R = 400
    G = N // R

    def body(x_ref, sum_ref, cnt_ref, rr_ref, wl_ref, wr_ref, ws_ref,
             bl_ref, misc_ref, out_ref):
        x = x_ref[...]
        agg = (sum_ref[0].astype(jnp.float32)
               + sum_ref[1].astype(jnp.float32))
        cnt = jnp.sum(cnt_ref[0], axis=0).reshape(R, 1)
        mean = agg / jnp.maximum(cnt, 1.0)
        z = (lax.dot_general(mean, wl_ref[...], (((1,), (1,)), ((), ())),
                             preferred_element_type=jnp.float32)
             + bl_ref[...]
             + lax.dot_general(x, wr_ref[...], (((1,), (1,)), ((), ())),
                               preferred_element_type=jnp.float32))
        h = jnp.maximum(z, 0.0) + x
        gnn = lax.dot_general(h, ws_ref[...], (((1,), (0,)), ((), ())),
                              preferred_element_type=jnp.float32)
        b_s = misc_ref[0, 0]
        a = 1.0 / (1.0 + jnp.exp(-misc_ref[0, 1]))
        out_ref[...] = a * rr_ref[...] + (1.0 - a) * (gnn + b_s)

    return pl.pallas_call(
        body,
        grid=(G,),
        in_specs=[
            pl.BlockSpec((R, D), lambda i: (i, 0)),
            pl.BlockSpec((NC, R, D), lambda i: (0, i, 0)),
            pl.BlockSpec((1, NW, R), lambda i: (i, 0, 0)),
            pl.BlockSpec((R, 1), lambda i: (i, 0)),
            pl.BlockSpec((H, D), lambda i: (0, 0)),
            pl.BlockSpec((H, D), lambda i: (0, 0)),
            pl.BlockSpec((H, 1), lambda i: (0, 0)),
            pl.BlockSpec((1, H), lambda i: (0, 0)),
            pl.BlockSpec((1, 2), lambda i: (0, 0)),
        ],
        out_specs=pl.BlockSpec((R, 1), lambda i: (i, 0)),
        out_shape=jax.ShapeDtypeStruct((N, 1), jnp.float32),
    )(x, sums, cnts, rr, W_l, W_r, W_sT, b_l2, misc)


def kernel(x, edge_index, reranker_scores, W_l, b_l, W_r, W_s, b_s, alpha):
    x_bf = x.astype(jnp.bfloat16)
    pad_iota = jnp.arange(EPAD, dtype=jnp.int32)
    src = (pad_iota % N).at[:E].set(edge_index[0])
    dst = (N + (pad_iota % (NPAD - N))).at[:E].set(edge_index[1])
    src_w = src.reshape(NW, NCHUNKS, CHUNK)
    dst_w = dst.reshape(NW, NCHUNKS, CHUNK)
    sums, cnts = _sc_segment_sum(x_bf, src_w, dst_w)
    cnts = cnts[:, :N].reshape(NW, N // 400, 400).transpose(1, 0, 2)
    misc = jnp.stack([b_s[0], alpha]).reshape(1, 2)
    out = _tc_combine(x, sums, cnts, reranker_scores.reshape(N, 1), W_l, W_r,
                      W_s.reshape(H, 1), b_l.reshape(1, H), misc)
    return out[:, 0]


# speedup vs baseline: 14.1020x; 1.2243x over previous
import functools

import jax
import jax.numpy as jnp
from jax import lax
from jax.experimental import pallas as pl
from jax.experimental.pallas import tpu as pltpu, tpu_sc as plsc

N = 10000
E = 320000
D = 128
H = 128

NC = 2
NS = 16
NW = NC * NS
CHUNK = 128
NCHUNKS = E // CHUNK
NSMALL = 28
CPW_LO = NCHUNKS // NW
CPW_HI = CPW_LO + 1
KB = 4
GROUPS = 76 // KB
NPAD = 10240
ROWS_PER_TILE = NPAD // NS
VPC = CHUNK // 16
RC = 2000
CG = N // RC


def _sc_segment_sum(x_bf, src_c, dst_c):
    mesh = plsc.VectorSubcoreMesh(
        core_axis_name="c", subcore_axis_name="s", num_cores=NC, num_subcores=NS
    )

    @functools.partial(
        pl.kernel,
        out_type=(
            jax.ShapeDtypeStruct((NC, NPAD, D), jnp.bfloat16),
            jax.ShapeDtypeStruct((CG, NW, RC), jnp.float32),
        ),
        mesh=mesh,
        compiler_params=pltpu.CompilerParams(needs_layout_passes=False,
                                             use_tc_tiling_on_sc=False),
        scratch_types=[
            pltpu.VMEM((CPW_HI, CHUNK), jnp.int32),
            pltpu.VMEM((CPW_HI, CHUNK), jnp.int32),
            [pltpu.VMEM((CHUNK, D), jnp.bfloat16)] * KB,
            pltpu.VMEM((CHUNK, D), jnp.bfloat16),
            pltpu.VMEM((NPAD,), jnp.float32),
            pltpu.VMEM_SHARED((NPAD, D), jnp.bfloat16),
            [pltpu.SemaphoreType.DMA] * KB,
        ],
    )
    def seg_kernel(x_hbm, src_hbm, dst_hbm, sum_out, cnt_out,
                   src_v, dst_v, bufs, zbuf, cnt_v, acc_sh, sems):
        c = lax.axis_index("c")
        s = lax.axis_index("s")
        wid = c * NS + s
        base = s * ROWS_PER_TILE
        is_big = (wid >= NSMALL).astype(jnp.int32)
        nw = CPW_LO + is_big
        start = jnp.where(is_big == 1,
                          NSMALL * CPW_LO + CPW_HI * (wid - NSMALL),
                          CPW_LO * wid)

        pltpu.sync_copy(src_hbm.at[pl.ds(start, CPW_HI)], src_v)
        pltpu.sync_copy(dst_hbm.at[pl.ds(start, CPW_HI)], dst_v)

        z32b = jnp.zeros((32,), jnp.bfloat16)

        def zero_zbuf(i, _):
            zbuf[i // (D // 32), pl.ds((i % (D // 32)) * 32, 32)] = z32b
            return 0
        lax.fori_loop(0, CHUNK * (D // 32), zero_zbuf, 0)

        z16 = jnp.zeros((16,), jnp.float32)

        def zero_cnt(i, _):
            cnt_v[pl.ds(i * 16, 16)] = z16
            return 0
        lax.fori_loop(0, NPAD // 16, zero_cnt, 0)

        for k in range(ROWS_PER_TILE // CHUNK):
            pltpu.sync_copy(zbuf, acc_sh.at[pl.ds(base + k * CHUNK, CHUNK)])

        plsc.subcore_barrier()

        ones16 = jnp.ones((16,), jnp.float32)

        def outer(gg, _):
            j0 = gg * KB
            for b in range(KB):
                pltpu.async_copy(x_hbm.at[src_v.at[j0 + b]], bufs[b], sems[b])
            for b in range(KB):
                for v in range(VPC):
                    idx = dst_v[j0 + b, pl.ds(v * 16, 16)]
                    plsc.addupdate_scatter(cnt_v, [idx], ones16)
            for b in range(KB):
                pltpu.make_async_copy(
                    x_hbm.at[src_v.at[j0 + b]], bufs[b], sems[b]).wait()
                pltpu.sync_copy(bufs[b], acc_sh.at[dst_v.at[j0 + b]], add=True)
            return 0
        lax.fori_loop(0, GROUPS, outer, 0)

        def tail(j, _):
            for v in range(VPC):
                idx = dst_v[j, pl.ds(v * 16, 16)]
                plsc.addupdate_scatter(cnt_v, [idx], ones16)
            pltpu.async_copy(x_hbm.at[src_v.at[j]], bufs[0], sems[0])
            pltpu.make_async_copy(x_hbm.at[src_v.at[j]], bufs[0], sems[0]).wait()
            pltpu.sync_copy(bufs[0], acc_sh.at[dst_v.at[j]], add=True)
            return 0
        lax.fori_loop(GROUPS * KB, nw, tail, 0)

        for g in range(CG):
            pltpu.sync_copy(cnt_v.at[pl.ds(g * RC, RC)], cnt_out.at[g, wid])

        plsc.subcore_barrier()

        pltpu.sync_copy(acc_sh.at[pl.ds(base, ROWS_PER_TILE)],
                        sum_out.at[c, pl.ds(base, ROWS_PER_TILE)])

    return seg_kernel(x_bf, src_c, dst_c)


def _tc_combine(x, sums, cnts, rr, W_l, W_r, W_sT, b_l2, misc):
    def body(x_ref, sum_ref, cnt_ref, rr_ref, wl_ref, wr_ref, ws_ref,
             bl_ref, misc_ref, out_ref):
        x = x_ref[...]
        agg = (sum_ref[0].astype(jnp.float32)
               + sum_ref[1].astype(jnp.float32))
        cnt = jnp.sum(cnt_ref[0], axis=0).reshape(RC, 1)
        mean = agg / jnp.maximum(cnt, 1.0)
        z = (lax.dot_general(mean, wl_ref[...], (((1,), (1,)), ((), ())),
                             preferred_element_type=jnp.float32)
             + bl_ref[...]
             + lax.dot_general(x, wr_ref[...], (((1,), (1,)), ((), ())),
                               preferred_element_type=jnp.float32))
        h = jnp.maximum(z, 0.0) + x
        gnn = lax.dot_general(h, ws_ref[...], (((1,), (0,)), ((), ())),
                              preferred_element_type=jnp.float32)
        b_s = misc_ref[0, 0]
        a = 1.0 / (1.0 + jnp.exp(-misc_ref[0, 1]))
        out_ref[...] = a * rr_ref[...] + (1.0 - a) * (gnn + b_s)

    return pl.pallas_call(
        body,
        grid=(CG,),
        in_specs=[
            pl.BlockSpec((RC, D), lambda i: (i, 0)),
            pl.BlockSpec((NC, RC, D), lambda i: (0, i, 0)),
            pl.BlockSpec((1, NW, RC), lambda i: (i, 0, 0)),
            pl.BlockSpec((RC, 1), lambda i: (i, 0)),
            pl.BlockSpec((H, D), lambda i: (0, 0)),
            pl.BlockSpec((H, D), lambda i: (0, 0)),
            pl.BlockSpec((H, 1), lambda i: (0, 0)),
            pl.BlockSpec((1, H), lambda i: (0, 0)),
            pl.BlockSpec((1, 2), lambda i: (0, 0)),
        ],
        out_specs=pl.BlockSpec((RC, 1), lambda i: (i, 0)),
        out_shape=jax.ShapeDtypeStruct((N, 1), jnp.float32),
    )(x, sums, cnts, rr, W_l, W_r, W_sT, b_l2, misc)


def kernel(x, edge_index, reranker_scores, W_l, b_l, W_r, W_s, b_s, alpha):
    x_bf = x.astype(jnp.bfloat16)
    src_c = edge_index[0].reshape(NCHUNKS, CHUNK)
    dst_c = edge_index[1].reshape(NCHUNKS, CHUNK)
    sums, cnts = _sc_segment_sum(x_bf, src_c, dst_c)
    misc = jnp.stack([b_s[0], alpha]).reshape(1, 2)
    out = _tc_combine(x, sums, cnts, reranker_scores.reshape(N, 1), W_l, W_r,
                      W_s.reshape(H, 1), b_l.reshape(1, H), misc)
    return out[:, 0]
